# baseline (device time: 48658 ns/iter reference)
import jax
import jax.numpy as jnp
from jax import lax
from jax.experimental import pallas as pl
from jax.experimental.pallas import tpu as pltpu

N_GLOBAL = 2048
EPS = 1e-5
C = 512


def kernel(x, gamma, beta):
    m, n_loc = x.shape
    nc = m // C

    def body(x_hbm, gamma_ref, beta_ref, out_ref, x_vmem,
             send_stats, recv_stats, in_sems, send_sems, recv_sems):
        my_x = lax.axis_index("x")
        my_y = lax.axis_index("y")
        peer = (my_x, 1 - my_y)

        with jax.named_scope("barrier"):
            barrier_sem = pltpu.get_barrier_semaphore()
            pl.semaphore_signal(
                barrier_sem, inc=1, device_id=peer,
                device_id_type=pl.DeviceIdType.MESH,
            )
            pl.semaphore_wait(barrier_sem, 1)

        copies_in = []
        for i in range(nc):
            cp = pltpu.make_async_copy(
                x_hbm.at[pl.ds(i * C, C), :],
                x_vmem.at[pl.ds(i * C, C), :],
                in_sems.at[i],
            )
            cp.start()
            copies_in.append(cp)

        rdmas = []
        for i in range(nc):
            with jax.named_scope(f"in_wait#c={i}"):
                copies_in[i].wait()
            with jax.named_scope(f"stats#c={i}"):
                xc = x_vmem[pl.ds(i * C, C), :]
                send_stats[i, :, 0:1] = jnp.sum(xc, axis=1, keepdims=True)
                send_stats[i, :, 1:2] = jnp.sum(xc * xc, axis=1, keepdims=True)
            rdma = pltpu.make_async_remote_copy(
                src_ref=send_stats.at[i],
                dst_ref=recv_stats.at[i],
                send_sem=send_sems.at[i],
                recv_sem=recv_sems.at[i],
                device_id=peer,
                device_id_type=pl.DeviceIdType.MESH,
            )
            rdma.start()
            rdmas.append(rdma)

        for i in range(nc):
            ds = pl.ds(i * C, C)
            with jax.named_scope(f"recv_wait#c={i}"):
                rdmas[i].wait_recv()
            with jax.named_scope(f"norm#c={i}"):
                tot1 = send_stats[i, :, 0:1] + recv_stats[i, :, 0:1]
                tot2 = send_stats[i, :, 1:2] + recv_stats[i, :, 1:2]
                mean_c = tot1 / N_GLOBAL
                var_c = tot2 / N_GLOBAL - mean_c * mean_c
                rstd_c = lax.rsqrt(var_c + EPS)
                out_ref[ds, :] = (
                    (x_vmem[ds, :] - mean_c) * rstd_c * gamma_ref[0:1, :]
                    + beta_ref[0:1, :]
                )

        with jax.named_scope("drain"):
            for i in range(nc):
                rdmas[i].wait_send()

    return pl.pallas_call(
        body,
        out_shape=jax.ShapeDtypeStruct((m, n_loc), jnp.float32),
        in_specs=[
            pl.BlockSpec(memory_space=pl.ANY),
            pl.BlockSpec(memory_space=pltpu.VMEM),
            pl.BlockSpec(memory_space=pltpu.VMEM),
        ],
        out_specs=pl.BlockSpec(memory_space=pltpu.VMEM),
        scratch_shapes=[
            pltpu.VMEM((m, n_loc), jnp.float32),
            pltpu.VMEM((nc, C, 2), jnp.float32),
            pltpu.VMEM((nc, C, 2), jnp.float32),
            pltpu.SemaphoreType.DMA((nc,)),
            pltpu.SemaphoreType.DMA((nc,)),
            pltpu.SemaphoreType.DMA((nc,)),
        ],
        compiler_params=pltpu.CompilerParams(
            collective_id=0, vmem_limit_bytes=48 * 1024 * 1024,
        ),
    )(x, gamma.reshape(1, n_loc), beta.reshape(1, n_loc))


# device time: 22693 ns/iter; 2.1442x vs baseline; 2.1442x over previous
import jax
import jax.numpy as jnp
from jax import lax
from jax.experimental import pallas as pl
from jax.experimental.pallas import tpu as pltpu

N_GLOBAL = 2048
EPS = 1e-5
C = 512


def kernel(x, gamma, beta):
    m, n_loc = x.shape
    nc = m // C

    def body(x_hbm, gamma_ref, beta_ref, out_ref, x_vmem,
             send_stats, recv_stats, in_sems, send_sems, recv_sems):
        my_x = lax.axis_index("x")
        my_y = lax.axis_index("y")
        peer = (my_x, 1 - my_y)

        with jax.named_scope("barrier"):
            barrier_sem = pltpu.get_barrier_semaphore()
            pl.semaphore_signal(
                barrier_sem, inc=1, device_id=peer,
                device_id_type=pl.DeviceIdType.MESH,
            )
            pl.semaphore_wait(barrier_sem, 1)

        copies_in = []
        for i in range(nc):
            cp = pltpu.make_async_copy(
                x_hbm.at[pl.ds(i * C, C), :],
                x_vmem.at[pl.ds(i * C, C), :],
                in_sems.at[i],
            )
            cp.start()
            copies_in.append(cp)

        hc = nc // 2
        rdmas = []
        for i in range(nc):
            h, j = divmod(i, hc)
            with jax.named_scope(f"in_wait#c={i}"):
                copies_in[i].wait()
            with jax.named_scope(f"stats#c={i}"):
                xc = x_vmem[pl.ds(i * C, C), :]
                send_stats[h, :, 2 * j:2 * j + 1] = jnp.sum(
                    xc, axis=1, keepdims=True)
                send_stats[h, :, 2 * j + 1:2 * j + 2] = jnp.sum(
                    xc * xc, axis=1, keepdims=True)
            if j == hc - 1:
                rdma = pltpu.make_async_remote_copy(
                    src_ref=send_stats.at[h],
                    dst_ref=recv_stats.at[h],
                    send_sem=send_sems.at[h],
                    recv_sem=recv_sems.at[h],
                    device_id=peer,
                    device_id_type=pl.DeviceIdType.MESH,
                )
                rdma.start()
                rdmas.append(rdma)

        for i in range(nc):
            h, j = divmod(i, hc)
            ds = pl.ds(i * C, C)
            if j == 0:
                with jax.named_scope(f"recv_wait#h={h}"):
                    rdmas[h].wait_recv()
            with jax.named_scope(f"norm#c={i}"):
                tot1 = (send_stats[h, :, 2 * j:2 * j + 1]
                        + recv_stats[h, :, 2 * j:2 * j + 1])
                tot2 = (send_stats[h, :, 2 * j + 1:2 * j + 2]
                        + recv_stats[h, :, 2 * j + 1:2 * j + 2])
                mean_c = tot1 / N_GLOBAL
                var_c = tot2 / N_GLOBAL - mean_c * mean_c
                rstd_c = lax.rsqrt(var_c + EPS)
                out_ref[ds, :] = (
                    (x_vmem[ds, :] - mean_c) * rstd_c * gamma_ref[0:1, :]
                    + beta_ref[0:1, :]
                )

        with jax.named_scope("drain"):
            for h in range(2):
                rdmas[h].wait_send()

    return pl.pallas_call(
        body,
        out_shape=jax.ShapeDtypeStruct((m, n_loc), jnp.float32),
        in_specs=[
            pl.BlockSpec(memory_space=pl.ANY),
            pl.BlockSpec(memory_space=pltpu.VMEM),
            pl.BlockSpec(memory_space=pltpu.VMEM),
        ],
        out_specs=pl.BlockSpec(memory_space=pltpu.VMEM),
        scratch_shapes=[
            pltpu.VMEM((m, n_loc), jnp.float32),
            pltpu.VMEM((2, C, nc), jnp.float32),
            pltpu.VMEM((2, C, nc), jnp.float32),
            pltpu.SemaphoreType.DMA((nc,)),
            pltpu.SemaphoreType.DMA((2,)),
            pltpu.SemaphoreType.DMA((2,)),
        ],
        compiler_params=pltpu.CompilerParams(collective_id=0),
    )(x, gamma.reshape(1, n_loc), beta.reshape(1, n_loc))


# device time: 22646 ns/iter; 2.1486x vs baseline; 1.0021x over previous
import jax
import jax.numpy as jnp
from jax import lax
from jax.experimental import pallas as pl
from jax.experimental.pallas import tpu as pltpu

N_GLOBAL = 2048
EPS = 1e-5
C = 512


def kernel(x, gamma, beta):
    m, n_loc = x.shape
    nc = m // C
    hc = nc // 2

    def body(x_hbm, gamma_hbm, beta_hbm, out_ref, x_vmem, gb_vmem,
             send_stats, recv_stats, in_sems, gb_sem, send_sems, recv_sems):
        my_x = lax.axis_index("x")
        my_y = lax.axis_index("y")
        peer = (my_x, 1 - my_y)

        copies_in = []
        for i in range(nc):
            cp = pltpu.make_async_copy(
                x_hbm.at[pl.ds(i * C, C), :],
                x_vmem.at[pl.ds(i * C, C), :],
                in_sems.at[i],
            )
            cp.start()
            copies_in.append(cp)
        g_cp = pltpu.make_async_copy(gamma_hbm, gb_vmem.at[0:1], gb_sem)
        g_cp.start()
        g_cp.wait()
        b_cp = pltpu.make_async_copy(beta_hbm, gb_vmem.at[1:2], gb_sem)
        b_cp.start()
        b_cp.wait()

        barrier_sem = pltpu.get_barrier_semaphore()
        pl.semaphore_signal(
            barrier_sem, inc=1, device_id=peer,
            device_id_type=pl.DeviceIdType.MESH,
        )

        rdmas = []
        for i in range(nc):
            h, j = divmod(i, hc)
            with jax.named_scope(f"in_wait#c={i}"):
                copies_in[i].wait()
            with jax.named_scope(f"stats#c={i}"):
                xc = x_vmem[pl.ds(i * C, C), :]
                send_stats[h, :, 2 * j:2 * j + 1] = jnp.sum(
                    xc, axis=1, keepdims=True)
                send_stats[h, :, 2 * j + 1:2 * j + 2] = jnp.sum(
                    xc * xc, axis=1, keepdims=True)
            if j == hc - 1:
                if h == 0:
                    with jax.named_scope("barrier_wait"):
                        pl.semaphore_wait(barrier_sem, 1)
                rdma = pltpu.make_async_remote_copy(
                    src_ref=send_stats.at[h],
                    dst_ref=recv_stats.at[h],
                    send_sem=send_sems.at[h],
                    recv_sem=recv_sems.at[h],
                    device_id=peer,
                    device_id_type=pl.DeviceIdType.MESH,
                )
                rdma.start()
                rdmas.append(rdma)

        for i in range(nc):
            h, j = divmod(i, hc)
            ds = pl.ds(i * C, C)
            if j == 0:
                with jax.named_scope(f"recv_wait#h={h}"):
                    rdmas[h].wait_recv()
            with jax.named_scope(f"norm#c={i}"):
                tot1 = (send_stats[h, :, 2 * j:2 * j + 1]
                        + recv_stats[h, :, 2 * j:2 * j + 1])
                tot2 = (send_stats[h, :, 2 * j + 1:2 * j + 2]
                        + recv_stats[h, :, 2 * j + 1:2 * j + 2])
                mean_c = tot1 / N_GLOBAL
                var_c = tot2 / N_GLOBAL - mean_c * mean_c
                rstd_c = lax.rsqrt(var_c + EPS)
                shift_c = -mean_c * rstd_c
                t = x_vmem[ds, :] * rstd_c + shift_c
                out_ref[ds, :] = t * gb_vmem[0:1, :] + gb_vmem[1:2, :]

        with jax.named_scope("drain"):
            for h in range(2):
                rdmas[h].wait_send()

    return pl.pallas_call(
        body,
        out_shape=jax.ShapeDtypeStruct((m, n_loc), jnp.float32),
        in_specs=[
            pl.BlockSpec(memory_space=pl.ANY),
            pl.BlockSpec(memory_space=pl.ANY),
            pl.BlockSpec(memory_space=pl.ANY),
        ],
        out_specs=pl.BlockSpec(memory_space=pltpu.VMEM),
        scratch_shapes=[
            pltpu.VMEM((m, n_loc), jnp.float32),
            pltpu.VMEM((2, n_loc), jnp.float32),
            pltpu.VMEM((2, C, nc), jnp.float32),
            pltpu.VMEM((2, C, nc), jnp.float32),
            pltpu.SemaphoreType.DMA((nc,)),
            pltpu.SemaphoreType.DMA,
            pltpu.SemaphoreType.DMA((2,)),
            pltpu.SemaphoreType.DMA((2,)),
        ],
        compiler_params=pltpu.CompilerParams(collective_id=0),
    )(x, gamma.reshape(1, n_loc), beta.reshape(1, n_loc))
